# Initial kernel scaffold; baseline (speedup 1.0000x reference)
import functools

import jax
import jax.numpy as jnp
from jax import lax
from jax.experimental import pallas as pl
from jax.experimental.pallas import tpu as pltpu
from jax.experimental.pallas import tpu_sc as plsc

N = 10000
E = 320000
D = 128
NG = 256
NP = 10240
CH = 128
NCH = 79
EPAD = 32 * NCH * CH
NPOOL = 320
PCH = 3
EPOOL = 32 * PCH * CH
BR = 512
G = NP // BR

_MESH = plsc.VectorSubcoreMesh(core_axis_name="c", subcore_axis_name="s")


def _make_sc_scatter(n_rows, n_chunks):
    rp = n_rows // 16

    @functools.partial(
        pl.kernel,
        out_type=jax.ShapeDtypeStruct((2, n_rows, 128), jnp.float32),
        mesh=_MESH,
        scratch_types=[
            pltpu.VMEM((n_chunks, CH), jnp.int32),
            pltpu.VMEM((n_chunks, CH), jnp.int32),
            pltpu.VMEM((CH, 128), jnp.float32),
            pltpu.VMEM_SHARED((n_rows, 128), jnp.float32),
            pltpu.SemaphoreType.DMA,
        ],
    )
    def k(y_hbm, src_hbm, dst_hbm, zeros_hbm, out_hbm, src_v, dst_v, rows_v,
          acc_sh, sem):
        c = lax.axis_index("c")
        s = lax.axis_index("s")
        wid = c * 16 + s
        pltpu.sync_copy(zeros_hbm.at[pl.ds(0, rp)], acc_sh.at[pl.ds(s * rp, rp)])
        pltpu.sync_copy(src_hbm.at[wid], src_v)
        pltpu.sync_copy(dst_hbm.at[wid], dst_v)
        plsc.subcore_barrier()

        def body(j, carry):
            pltpu.async_copy(y_hbm.at[src_v.at[j]], rows_v, sem).wait()
            pltpu.sync_copy(rows_v, acc_sh.at[dst_v.at[j]], add=True)
            return carry

        lax.fori_loop(0, n_chunks, body, 0)
        plsc.subcore_barrier()
        pltpu.sync_copy(acc_sh.at[pl.ds(s * rp, rp)],
                        out_hbm.at[c].at[pl.ds(s * rp, rp)])

    return k


def _make_sc_hist(n_vals, hsize):
    per_tile = n_vals // 32
    n_vec = per_tile // 16

    @functools.partial(
        pl.kernel,
        out_type=jax.ShapeDtypeStruct((32, hsize), jnp.float32),
        mesh=_MESH,
        scratch_types=[
            pltpu.VMEM((per_tile,), jnp.int32),
            pltpu.VMEM((hsize,), jnp.float32),
        ],
    )
    def k(vals_hbm, out_hbm, vals_v, hist_v):
        c = lax.axis_index("c")
        s = lax.axis_index("s")
        wid = c * 16 + s
        pltpu.sync_copy(vals_hbm.at[pl.ds(wid * per_tile, per_tile)], vals_v)

        def zbody(j, carry):
            hist_v[pl.ds(j * 16, 16)] = jnp.zeros((16,), jnp.float32)
            return carry

        lax.fori_loop(0, hsize // 16, zbody, 0)
        ones = jnp.ones((16,), jnp.float32)

        def body(j, carry):
            idx = vals_v[pl.ds(j * 16, 16)]
            plsc.addupdate_scatter(hist_v, [idx], ones)
            return carry

        lax.fori_loop(0, n_vec, body, 0)
        pltpu.sync_copy(hist_v, out_hbm.at[wid])

    return k


_sc_scatter_edges = _make_sc_scatter(NP, NCH)
_sc_scatter_pool = _make_sc_scatter(NPOOL, PCH)
_sc_hist_deg = _make_sc_hist(E, NP)
_sc_hist_cnt = _make_sc_hist(NP, NPOOL)


def _tc_dinv(hist3):
    def body(h_ref, d_ref):
        i = pl.program_id(0)
        deg = jnp.sum(h_ref[...], axis=0) + 1.0
        row = i * BR + lax.broadcasted_iota(jnp.int32, (BR, 1), 0)
        d_ref[...] = jnp.where(row < N, lax.rsqrt(deg), 0.0)

    return pl.pallas_call(
        body,
        grid=(G,),
        in_specs=[pl.BlockSpec((32, BR, 1), lambda i: (0, i, 0))],
        out_specs=pl.BlockSpec((BR, 1), lambda i: (i, 0)),
        out_shape=jax.ShapeDtypeStruct((NP, 1), jnp.float32),
    )(hist3)


def _tc_input_proj(xp, Wp, b):
    def body(x_ref, w_ref, b_ref, o_ref):
        o_ref[...] = jnp.maximum(
            jnp.dot(x_ref[...], w_ref[...],
                    preferred_element_type=jnp.float32) + b_ref[...], 0.0)

    return pl.pallas_call(
        body,
        grid=(G,),
        in_specs=[pl.BlockSpec((BR, 256), lambda i: (i, 0)),
                  pl.BlockSpec((256, 128), lambda i: (0, 0)),
                  pl.BlockSpec((1, 128), lambda i: (0, 0))],
        out_specs=pl.BlockSpec((BR, 128), lambda i: (i, 0)),
        out_shape=jax.ShapeDtypeStruct((NP, 128), jnp.float32),
    )(xp, Wp, b)


def _tc_matmul_scale(h, W, dinv):
    def body(h_ref, w_ref, d_ref, y_ref):
        y_ref[...] = d_ref[...] * jnp.dot(
            h_ref[...], w_ref[...], preferred_element_type=jnp.float32)

    return pl.pallas_call(
        body,
        grid=(G,),
        in_specs=[pl.BlockSpec((BR, 128), lambda i: (i, 0)),
                  pl.BlockSpec((128, 128), lambda i: (0, 0)),
                  pl.BlockSpec((BR, 1), lambda i: (i, 0))],
        out_specs=pl.BlockSpec((BR, 128), lambda i: (i, 0)),
        out_shape=jax.ShapeDtypeStruct((NP, 128), jnp.float32),
    )(h, W, dinv)


def _tc_combine_stats(acc, y, dinv):
    def body(a_ref, y_ref, d_ref, o_ref, st_ref):
        i = pl.program_id(0)
        o = d_ref[...] * (a_ref[0] + a_ref[1] + y_ref[...])
        o_ref[...] = o
        s1 = jnp.sum(o, axis=0, keepdims=True)
        s2 = jnp.sum(o * o, axis=0, keepdims=True)
        part = jnp.concatenate(
            [s1, s2, jnp.zeros((6, 128), jnp.float32)], axis=0)

        @pl.when(i == 0)
        def _():
            st_ref[...] = part

        @pl.when(i > 0)
        def _():
            st_ref[...] += part

    return pl.pallas_call(
        body,
        grid=(G,),
        in_specs=[pl.BlockSpec((2, BR, 128), lambda i: (0, i, 0)),
                  pl.BlockSpec((BR, 128), lambda i: (i, 0)),
                  pl.BlockSpec((BR, 1), lambda i: (i, 0))],
        out_specs=[pl.BlockSpec((BR, 128), lambda i: (i, 0)),
                   pl.BlockSpec((8, 128), lambda i: (0, 0))],
        out_shape=[jax.ShapeDtypeStruct((NP, 128), jnp.float32),
                   jax.ShapeDtypeStruct((8, 128), jnp.float32)],
    )(acc, y, dinv)


def _tc_apply_bn(out, st, g, b):
    def body(o_ref, st_ref, g_ref, b_ref, h_ref):
        mean = st_ref[0:1, :] * (1.0 / N)
        ex2 = st_ref[1:2, :] * (1.0 / N)
        var = ex2 - mean * mean
        rstd = lax.rsqrt(var + 1e-5)
        h_ref[...] = jnp.maximum(
            (o_ref[...] - mean) * rstd * g_ref[...] + b_ref[...], 0.0)

    return pl.pallas_call(
        body,
        grid=(G,),
        in_specs=[pl.BlockSpec((BR, 128), lambda i: (i, 0)),
                  pl.BlockSpec((8, 128), lambda i: (0, 0)),
                  pl.BlockSpec((1, 128), lambda i: (0, 0)),
                  pl.BlockSpec((1, 128), lambda i: (0, 0))],
        out_specs=pl.BlockSpec((BR, 128), lambda i: (i, 0)),
        out_shape=jax.ShapeDtypeStruct((NP, 128), jnp.float32),
    )(out, st, g, b)


def _tc_predictor(pool_acc, cnt3, W1, b1, W2, b2):
    def body(a_ref, c_ref, w1_ref, b1_ref, w2_ref, b2_ref, p_ref):
        cnt = jnp.sum(c_ref[...], axis=0)
        cnt = jnp.maximum(cnt[:NG], 1.0)
        emb = (a_ref[0, :NG, :] + a_ref[1, :NG, :]) / cnt
        hid = jnp.maximum(
            jnp.dot(emb, w1_ref[...],
                    preferred_element_type=jnp.float32) + b1_ref[...], 0.0)
        p_ref[...] = jnp.dot(
            hid, w2_ref[...], preferred_element_type=jnp.float32) + b2_ref[...]

    return pl.pallas_call(
        body,
        in_specs=[pl.BlockSpec((2, NPOOL, 128), lambda: (0, 0, 0)),
                  pl.BlockSpec((32, NPOOL, 1), lambda: (0, 0, 0)),
                  pl.BlockSpec((128, 128), lambda: (0, 0)),
                  pl.BlockSpec((1, 128), lambda: (0, 0)),
                  pl.BlockSpec((128, 19), lambda: (0, 0)),
                  pl.BlockSpec((1, 19), lambda: (0, 0))],
        out_specs=pl.BlockSpec((NG, 19), lambda: (0, 0)),
        out_shape=jax.ShapeDtypeStruct((NG, 19), jnp.float32),
    )(pool_acc, cnt3, W1, b1, W2, b2)


def kernel(x, pos, edge_index, batch, lin_W, lin_b, conv_W, conv_b, bn_g,
           bn_b, pred_W1, pred_b1, pred_W2, pred_b2):
    del conv_b
    src = edge_index[0].astype(jnp.int32)
    dst = edge_index[1].astype(jnp.int32)
    bat = batch.astype(jnp.int32)
    src_p = jnp.concatenate(
        [src, jnp.zeros((EPAD - E,), jnp.int32)]).reshape(32, NCH, CH)
    dst_p = jnp.concatenate(
        [dst, jnp.full((EPAD - E,), N, jnp.int32)]).reshape(32, NCH, CH)
    bat_pad = jnp.concatenate([bat, jnp.full((NP - N,), NG, jnp.int32)])
    psrc = jnp.concatenate(
        [jnp.arange(N, dtype=jnp.int32),
         jnp.zeros((EPOOL - N,), jnp.int32)]).reshape(32, PCH, CH)
    pdst = jnp.concatenate(
        [bat, jnp.full((EPOOL - N,), NG, jnp.int32)]).reshape(32, PCH, CH)
    xp = (jnp.zeros((NP, 256), jnp.float32)
          .at[:N, :D].set(x).at[:N, D:D + 3].set(pos))
    Wp = jnp.zeros((256, 128), jnp.float32).at[:D + 3].set(lin_W)
    zeros_sc = jnp.zeros((NP // 16, 128), jnp.float32)

    deg_hist = _sc_hist_deg(dst)
    cnt_hist = _sc_hist_cnt(bat_pad)
    dinv = _tc_dinv(deg_hist.reshape(32, NP, 1))
    h = _tc_input_proj(xp, Wp, lin_b.reshape(1, 128))
    for i in range(4):
        y = _tc_matmul_scale(h, conv_W[i], dinv)
        acc = _sc_scatter_edges(y, src_p, dst_p, zeros_sc)
        out, st = _tc_combine_stats(acc, y, dinv)
        h = _tc_apply_bn(out, st, bn_g[i].reshape(1, 128),
                         bn_b[i].reshape(1, 128))
    pool = _sc_scatter_pool(h, psrc, pdst, zeros_sc)
    return _tc_predictor(pool, cnt_hist.reshape(32, NPOOL, 1),
                         pred_W1, pred_b1.reshape(1, 128),
                         pred_W2, pred_b2.reshape(1, 19))


# speedup vs baseline: 7.5552x; 7.5552x over previous
import functools

import jax
import jax.numpy as jnp
from jax import lax
from jax.experimental import pallas as pl
from jax.experimental.pallas import tpu as pltpu
from jax.experimental.pallas import tpu_sc as plsc

N = 10000
E = 320000
D = 128
NG = 256
NP = 10240
CH = 128
NCH = 79
EPAD = 32 * NCH * CH
NPOOL = 512
PCH = 3
EPOOL = 32 * PCH * CH
BR = 512
G = NP // BR

_MESH = plsc.VectorSubcoreMesh(core_axis_name="c", subcore_axis_name="s")


def _make_sc_scatter(n_rows, n_chunks):
    rp = n_rows // 16

    @functools.partial(
        pl.kernel,
        out_type=jax.ShapeDtypeStruct((2, n_rows, 128), jnp.float32),
        mesh=_MESH,
        scratch_types=[
            pltpu.VMEM((n_chunks, CH), jnp.int32),
            pltpu.VMEM((n_chunks, CH), jnp.int32),
            pltpu.VMEM((CH, 128), jnp.float32),
            pltpu.VMEM_SHARED((n_rows, 128), jnp.float32),
            pltpu.SemaphoreType.DMA,
        ],
        compiler_params=pltpu.CompilerParams(needs_layout_passes=False),
    )
    def k(y_hbm, src_hbm, dst_hbm, zeros_hbm, out_hbm, src_v, dst_v, rows_v,
          acc_sh, sem):
        c = lax.axis_index("c")
        s = lax.axis_index("s")
        wid = c * 16 + s
        pltpu.sync_copy(zeros_hbm.at[pl.ds(0, rp)], acc_sh.at[pl.ds(s * rp, rp)])
        pltpu.sync_copy(src_hbm.at[wid], src_v)
        pltpu.sync_copy(dst_hbm.at[wid], dst_v)
        plsc.subcore_barrier()

        def body(j, carry):
            pltpu.async_copy(y_hbm.at[src_v.at[j]], rows_v, sem).wait()
            pltpu.sync_copy(rows_v, acc_sh.at[dst_v.at[j]], add=True)
            return carry

        lax.fori_loop(0, n_chunks, body, 0)
        plsc.subcore_barrier()
        pltpu.sync_copy(acc_sh.at[pl.ds(s * rp, rp)],
                        out_hbm.at[c].at[pl.ds(s * rp, rp)])

    return k


def _make_sc_hist(n_vals, hsize):
    per_tile = n_vals // 32
    n_vec = per_tile // 16

    @functools.partial(
        pl.kernel,
        out_type=jax.ShapeDtypeStruct((32, hsize), jnp.float32),
        mesh=_MESH,
        scratch_types=[
            pltpu.VMEM((per_tile,), jnp.int32),
            pltpu.VMEM((hsize,), jnp.float32),
        ],
        compiler_params=pltpu.CompilerParams(needs_layout_passes=False),
    )
    def k(vals_hbm, out_hbm, vals_v, hist_v):
        c = lax.axis_index("c")
        s = lax.axis_index("s")
        wid = c * 16 + s
        pltpu.sync_copy(vals_hbm.at[pl.ds(wid * per_tile, per_tile)], vals_v)

        def zbody(j, carry):
            hist_v[pl.ds(j * 16, 16)] = jnp.zeros((16,), jnp.float32)
            return carry

        lax.fori_loop(0, hsize // 16, zbody, 0)
        ones = jnp.ones((16,), jnp.float32)

        def body(j, carry):
            idx = vals_v[pl.ds(j * 16, 16)]
            plsc.addupdate_scatter(hist_v, [idx], ones)
            return carry

        lax.fori_loop(0, n_vec, body, 0)
        pltpu.sync_copy(hist_v, out_hbm.at[wid])

    return k


_sc_scatter_edges = _make_sc_scatter(NP, NCH)
_sc_scatter_pool = _make_sc_scatter(NPOOL, PCH)
_sc_hist_deg = _make_sc_hist(E, NP)
_sc_hist_cnt = _make_sc_hist(NP, NPOOL)


def _tc_dinv(hist3):
    def body(h_ref, d_ref):
        i = pl.program_id(0)
        deg = jnp.sum(h_ref[...], axis=0) + 1.0
        row = i * BR + lax.broadcasted_iota(jnp.int32, (BR, 1), 0)
        d_ref[...] = jnp.where(row < N, lax.rsqrt(deg), 0.0)

    return pl.pallas_call(
        body,
        grid=(G,),
        in_specs=[pl.BlockSpec((32, BR, 1), lambda i: (0, i, 0))],
        out_specs=pl.BlockSpec((BR, 1), lambda i: (i, 0)),
        out_shape=jax.ShapeDtypeStruct((NP, 1), jnp.float32),
    )(hist3)


def _tc_input_proj(xp, Wp, b):
    def body(x_ref, w_ref, b_ref, o_ref):
        o_ref[...] = jnp.maximum(
            jnp.dot(x_ref[...], w_ref[...],
                    preferred_element_type=jnp.float32) + b_ref[...], 0.0)

    return pl.pallas_call(
        body,
        grid=(G,),
        in_specs=[pl.BlockSpec((BR, 256), lambda i: (i, 0)),
                  pl.BlockSpec((256, 128), lambda i: (0, 0)),
                  pl.BlockSpec((1, 128), lambda i: (0, 0))],
        out_specs=pl.BlockSpec((BR, 128), lambda i: (i, 0)),
        out_shape=jax.ShapeDtypeStruct((NP, 128), jnp.float32),
    )(xp, Wp, b)


def _tc_matmul_scale(h, W, dinv):
    def body(h_ref, w_ref, d_ref, y_ref):
        y_ref[...] = d_ref[...] * jnp.dot(
            h_ref[...], w_ref[...], preferred_element_type=jnp.float32)

    return pl.pallas_call(
        body,
        grid=(G,),
        in_specs=[pl.BlockSpec((BR, 128), lambda i: (i, 0)),
                  pl.BlockSpec((128, 128), lambda i: (0, 0)),
                  pl.BlockSpec((BR, 1), lambda i: (i, 0))],
        out_specs=pl.BlockSpec((BR, 128), lambda i: (i, 0)),
        out_shape=jax.ShapeDtypeStruct((NP, 128), jnp.float32),
    )(h, W, dinv)


def _tc_combine_stats(acc, y, dinv):
    def body(a_ref, y_ref, d_ref, o_ref, st_ref):
        i = pl.program_id(0)
        o = d_ref[...] * (a_ref[0] + a_ref[1] + y_ref[...])
        o_ref[...] = o
        s1 = jnp.sum(o, axis=0, keepdims=True)
        s2 = jnp.sum(o * o, axis=0, keepdims=True)
        part = jnp.concatenate(
            [s1, s2, jnp.zeros((6, 128), jnp.float32)], axis=0)

        @pl.when(i == 0)
        def _():
            st_ref[...] = part

        @pl.when(i > 0)
        def _():
            st_ref[...] += part

    return pl.pallas_call(
        body,
        grid=(G,),
        in_specs=[pl.BlockSpec((2, BR, 128), lambda i: (0, i, 0)),
                  pl.BlockSpec((BR, 128), lambda i: (i, 0)),
                  pl.BlockSpec((BR, 1), lambda i: (i, 0))],
        out_specs=[pl.BlockSpec((BR, 128), lambda i: (i, 0)),
                   pl.BlockSpec((8, 128), lambda i: (0, 0))],
        out_shape=[jax.ShapeDtypeStruct((NP, 128), jnp.float32),
                   jax.ShapeDtypeStruct((8, 128), jnp.float32)],
    )(acc, y, dinv)


def _tc_apply_bn(out, st, g, b):
    def body(o_ref, st_ref, g_ref, b_ref, h_ref):
        mean = st_ref[0:1, :] * (1.0 / N)
        ex2 = st_ref[1:2, :] * (1.0 / N)
        var = ex2 - mean * mean
        rstd = lax.rsqrt(var + 1e-5)
        h_ref[...] = jnp.maximum(
            (o_ref[...] - mean) * rstd * g_ref[...] + b_ref[...], 0.0)

    return pl.pallas_call(
        body,
        grid=(G,),
        in_specs=[pl.BlockSpec((BR, 128), lambda i: (i, 0)),
                  pl.BlockSpec((8, 128), lambda i: (0, 0)),
                  pl.BlockSpec((1, 128), lambda i: (0, 0)),
                  pl.BlockSpec((1, 128), lambda i: (0, 0))],
        out_specs=pl.BlockSpec((BR, 128), lambda i: (i, 0)),
        out_shape=jax.ShapeDtypeStruct((NP, 128), jnp.float32),
    )(out, st, g, b)


def _tc_predictor(pool_acc, cnt3, W1, b1, W2, b2):
    def body(a_ref, c_ref, w1_ref, b1_ref, w2_ref, b2_ref, p_ref):
        cnt = jnp.sum(c_ref[...], axis=0)
        cnt = jnp.maximum(cnt[:NG], 1.0)
        emb = (a_ref[0, :NG, :] + a_ref[1, :NG, :]) / cnt
        hid = jnp.maximum(
            jnp.dot(emb, w1_ref[...],
                    preferred_element_type=jnp.float32) + b1_ref[...], 0.0)
        p_ref[...] = jnp.dot(
            hid, w2_ref[...], preferred_element_type=jnp.float32) + b2_ref[...]

    return pl.pallas_call(
        body,
        in_specs=[pl.BlockSpec((2, NPOOL, 128), lambda: (0, 0, 0)),
                  pl.BlockSpec((32, NPOOL, 1), lambda: (0, 0, 0)),
                  pl.BlockSpec((128, 128), lambda: (0, 0)),
                  pl.BlockSpec((1, 128), lambda: (0, 0)),
                  pl.BlockSpec((128, 19), lambda: (0, 0)),
                  pl.BlockSpec((1, 19), lambda: (0, 0))],
        out_specs=pl.BlockSpec((NG, 19), lambda: (0, 0)),
        out_shape=jax.ShapeDtypeStruct((NG, 19), jnp.float32),
    )(pool_acc, cnt3, W1, b1, W2, b2)


def kernel(x, pos, edge_index, batch, lin_W, lin_b, conv_W, conv_b, bn_g,
           bn_b, pred_W1, pred_b1, pred_W2, pred_b2):
    del conv_b
    src = edge_index[0].astype(jnp.int32)
    dst = edge_index[1].astype(jnp.int32)
    bat = batch.astype(jnp.int32)
    src_p = jnp.concatenate(
        [src, jnp.zeros((EPAD - E,), jnp.int32)]).reshape(32, NCH, CH)
    dst_p = jnp.concatenate(
        [dst, jnp.full((EPAD - E,), N, jnp.int32)]).reshape(32, NCH, CH)
    bat_pad = jnp.concatenate([bat, jnp.full((NP - N,), NG, jnp.int32)])
    psrc = jnp.concatenate(
        [jnp.arange(N, dtype=jnp.int32),
         jnp.zeros((EPOOL - N,), jnp.int32)]).reshape(32, PCH, CH)
    pdst = jnp.concatenate(
        [bat, jnp.full((EPOOL - N,), NG, jnp.int32)]).reshape(32, PCH, CH)
    xp = (jnp.zeros((NP, 256), jnp.float32)
          .at[:N, :D].set(x).at[:N, D:D + 3].set(pos))
    Wp = jnp.zeros((256, 128), jnp.float32).at[:D + 3].set(lin_W)
    zeros_sc = jnp.zeros((NP // 16, 128), jnp.float32)

    deg_hist = _sc_hist_deg(dst)
    cnt_hist = _sc_hist_cnt(bat_pad)
    dinv = _tc_dinv(deg_hist.reshape(32, NP, 1))
    h = _tc_input_proj(xp, Wp, lin_b.reshape(1, 128))
    for i in range(4):
        y = _tc_matmul_scale(h, conv_W[i], dinv)
        acc = _sc_scatter_edges(y, src_p, dst_p, zeros_sc)
        out, st = _tc_combine_stats(acc, y, dinv)
        h = _tc_apply_bn(out, st, bn_g[i].reshape(1, 128),
                         bn_b[i].reshape(1, 128))
    pool = _sc_scatter_pool(h, psrc, pdst, zeros_sc)
    return _tc_predictor(pool, cnt_hist.reshape(32, NPOOL, 1),
                         pred_W1, pred_b1.reshape(1, 128),
                         pred_W2, pred_b2.reshape(1, 19))
